# Initial kernel scaffold; baseline (speedup 1.0000x reference)
import jax
import jax.numpy as jnp
from jax.experimental import pallas as pl


def kernel(x_prev, rand_node_features, t_idx_per_node, edge_index, W_enc1, b_enc1, W_enc2, b_enc2, W_msg, b_msg, W_upd1, b_upd1, W_upd2, b_upd2, W_dec1, b_dec1, W_dec2, b_dec2, W_head1, b_head1, W_head2, b_head2):
    raise NotImplementedError("write your pallas kernel here")


# speedup vs baseline: 6.3447x; 6.3447x over previous
import functools

import jax
import jax.numpy as jnp
from jax import lax
from jax.experimental import pallas as pl
from jax.experimental.pallas import tpu as pltpu
from jax.experimental.pallas import tpu_sc as plsc

N = 100000
E = 1600000
H = 48
EMB = 32
TMAX = 100
NMP = 8
NRAND = 5
NBERN = 2

N2 = 102400
BN = 1600
GRID = N2 // BN

FG = 16
NG = 3
NC = 2
NS = 16
SUB = 125
NSUB = 8
EROWS = E // SUB
ROWS_PT = EROWS // (NC * NS)
NCH = ROWS_PT // NSUB
ACC_PT = N2 // NS
ZB = 1280


def _sc_segsum_body(hw0, hw1, hw2, send2, recv2, zsrc, out,
                    idx_s, idx_r, rows, acc, sem):
    c = lax.axis_index("c")
    s = lax.axis_index("s")
    wid = c * NS + s
    row_base = wid * ROWS_PT

    hws = (hw0, hw1, hw2)
    for g in range(NG):
        hw = hws[g]
        for z in range(ACC_PT // ZB):
            pltpu.sync_copy(zsrc, acc.at[pl.ds(s * ACC_PT + z * ZB, ZB)])
        plsc.subcore_barrier()

        def _chunk(i, _, hw=hw):
            r0 = row_base + i * NSUB
            pltpu.sync_copy(send2.at[pl.ds(r0, NSUB)], idx_s)
            pltpu.sync_copy(recv2.at[pl.ds(r0, NSUB)], idx_r)
            for j in range(NSUB):
                pltpu.make_async_copy(hw.at[idx_s.at[j]], rows.at[j], sem).start()
            for j in range(NSUB):
                pltpu.make_async_copy(hw.at[idx_s.at[j]], rows.at[j], sem).wait()
            for j in range(NSUB):
                pltpu.sync_copy(rows.at[j], acc.at[idx_r.at[j]], add=True)
            return 0
        lax.fori_loop(0, NCH, _chunk, 0)
        plsc.subcore_barrier()

        pltpu.sync_copy(
            acc.at[pl.ds(s * ACC_PT, ACC_PT)],
            out.at[g, pl.ds(c * N2 + s * ACC_PT, ACC_PT)])
        plsc.subcore_barrier()


_sc_segsum = functools.partial(
    pl.kernel,
    out_type=jax.ShapeDtypeStruct((NG, NC * N2, FG), jnp.float32),
    mesh=plsc.VectorSubcoreMesh(core_axis_name="c", subcore_axis_name="s"),
    compiler_params=pltpu.CompilerParams(use_tc_tiling_on_sc=False),
    scratch_types=[
        pltpu.VMEM((NSUB, SUB), jnp.int32),
        pltpu.VMEM((NSUB, SUB), jnp.int32),
        pltpu.VMEM((NSUB, SUB, FG), jnp.float32),
        pltpu.VMEM_SHARED((N2, FG), jnp.float32),
        pltpu.SemaphoreType.DMA,
    ],
)(_sc_segsum_body)


def _full(shape):
    return pl.BlockSpec(shape, lambda i: (0,) * len(shape))


def _rows(bn, width):
    return pl.BlockSpec((bn, width), lambda i: (i, 0))


def _enc_body(xp, t, rnd, div, wa0, wa1, wb1, wb2, wc, be1, we2, be2, wm, bm,
              h_out, hw0_out, hw1_out, hw2_out):
    tf = t[...].astype(jnp.float32)
    arg = tf * div[...]
    sn, cs = jnp.sin(arg), jnp.cos(arg)
    xsel = jnp.where(xp[...] == 0, wa0[...], wa1[...])
    u = (xsel + jnp.dot(sn, wb1[...], preferred_element_type=jnp.float32)
         + jnp.dot(cs, wb2[...], preferred_element_type=jnp.float32)
         + jnp.dot(rnd[...], wc[...], preferred_element_type=jnp.float32)
         + be1[...])
    h = jnp.maximum(u, 0.0)
    h = jnp.maximum(jnp.dot(h, we2[...], preferred_element_type=jnp.float32)
                    + be2[...], 0.0)
    hw = jnp.dot(h, wm[...], preferred_element_type=jnp.float32) + bm[...]
    h_out[...] = h
    hw0_out[...] = hw[:, 0:FG]
    hw1_out[...] = hw[:, FG:2 * FG]
    hw2_out[...] = hw[:, 2 * FG:3 * FG]


_enc_call = pl.pallas_call(
    _enc_body,
    grid=(GRID,),
    in_specs=[
        _rows(BN, 1), _rows(BN, 1), _rows(BN, NRAND), _full((1, FG)),
        _full((1, H)), _full((1, H)),
        _full((FG, H)), _full((FG, H)), _full((NRAND, H)), _full((1, H)),
        _full((H, H)), _full((1, H)), _full((H, H)), _full((1, H)),
    ],
    out_specs=[_rows(BN, H), _rows(BN, FG), _rows(BN, FG), _rows(BN, FG)],
    out_shape=[
        jax.ShapeDtypeStruct((N2, H), jnp.float32),
        jax.ShapeDtypeStruct((N2, FG), jnp.float32),
        jax.ShapeDtypeStruct((N2, FG), jnp.float32),
        jax.ShapeDtypeStruct((N2, FG), jnp.float32),
    ],
)


def _upd_body(with_hw, h, p0, p1, wua, wub0, wub1, wub2, bu1, wu2, bu2, wm, bm,
              *outs):
    agg0 = p0[0, :, :] + p1[0, :, :]
    agg1 = p0[1, :, :] + p1[1, :, :]
    agg2 = p0[2, :, :] + p1[2, :, :]
    u = (jnp.dot(h[...], wua[...], preferred_element_type=jnp.float32)
         + jnp.dot(agg0, wub0[...], preferred_element_type=jnp.float32)
         + jnp.dot(agg1, wub1[...], preferred_element_type=jnp.float32)
         + jnp.dot(agg2, wub2[...], preferred_element_type=jnp.float32)
         + bu1[...])
    u = jnp.maximum(u, 0.0)
    hn = jnp.maximum(jnp.dot(u, wu2[...], preferred_element_type=jnp.float32)
                     + bu2[...], 0.0)
    outs[0][...] = hn
    if with_hw:
        hw = jnp.dot(hn, wm[...], preferred_element_type=jnp.float32) + bm[...]
        outs[1][...] = hw[:, 0:FG]
        outs[2][...] = hw[:, FG:2 * FG]
        outs[3][...] = hw[:, 2 * FG:3 * FG]


def _make_upd(with_hw):
    n_out = 4 if with_hw else 1
    return pl.pallas_call(
        functools.partial(_upd_body, with_hw),
        grid=(GRID,),
        in_specs=[
            _rows(BN, H),
            pl.BlockSpec((NG, BN, FG), lambda i: (0, i, 0)),
            pl.BlockSpec((NG, BN, FG), lambda i: (0, N2 // BN + i, 0)),
            _full((H, H)), _full((FG, H)), _full((FG, H)), _full((FG, H)),
            _full((1, H)), _full((H, H)), _full((1, H)),
            _full((H, H)), _full((1, H)),
        ],
        out_specs=([_rows(BN, H)] + [_rows(BN, FG)] * 3)[:n_out],
        out_shape=([jax.ShapeDtypeStruct((N2, H), jnp.float32)]
                   + [jax.ShapeDtypeStruct((N2, FG), jnp.float32)] * 3)[:n_out],
    )


_upd_hw = _make_upd(True)
_upd_last = _make_upd(False)


def _dec_body(h, wd1, bd1, wd2, bd2, wh1, bh1, wh2, bh2, out):
    d = jnp.maximum(jnp.dot(h[...], wd1[...], preferred_element_type=jnp.float32)
                    + bd1[...], 0.0)
    d = jnp.dot(d, wd2[...], preferred_element_type=jnp.float32) + bd2[...]
    e = jnp.maximum(jnp.dot(d, wh1[...], preferred_element_type=jnp.float32)
                    + bh1[...], 0.0)
    out[...] = (jnp.dot(e, wh2[...], preferred_element_type=jnp.float32)
                + bh2[...])


_dec_call = pl.pallas_call(
    _dec_body,
    grid=(GRID,),
    in_specs=[
        _rows(BN, H),
        _full((H, H)), _full((1, H)), _full((H, H)), _full((1, H)),
        _full((H, H)), _full((1, H)), _full((H, NBERN)), _full((1, NBERN)),
    ],
    out_specs=_rows(BN, NBERN),
    out_shape=jax.ShapeDtypeStruct((N2, NBERN), jnp.float32),
)


def kernel(x_prev, rand_node_features, t_idx_per_node, edge_index,
           W_enc1, b_enc1, W_enc2, b_enc2, W_msg, b_msg,
           W_upd1, b_upd1, W_upd2, b_upd2, W_dec1, b_dec1, W_dec2, b_dec2,
           W_head1, b_head1, W_head2, b_head2):
    ei = edge_index.astype(jnp.int32)
    send2 = ei[0].reshape(EROWS, SUB)
    recv2 = ei[1].reshape(EROWS, SUB)

    pad = ((0, N2 - N), (0, 0))
    xp = jnp.pad(x_prev, pad)
    tp = jnp.pad(t_idx_per_node, pad)
    rp = jnp.pad(rand_node_features, pad)

    div = jnp.exp(jnp.arange(0, EMB, 2, dtype=jnp.float32)
                  * (-jnp.log(float(TMAX)) / EMB)).reshape(1, FG)

    r2 = lambda b: b.reshape(1, -1)
    h, hw0, hw1, hw2 = _enc_call(
        xp, tp, rp, div,
        W_enc1[0:1], W_enc1[1:2],
        W_enc1[2:2 + FG], W_enc1[2 + FG:2 + EMB], W_enc1[2 + EMB:],
        r2(b_enc1), W_enc2, r2(b_enc2), W_msg, r2(b_msg))

    wua = W_upd1[:H]
    wub0 = W_upd1[H:H + FG]
    wub1 = W_upd1[H + FG:H + 2 * FG]
    wub2 = W_upd1[H + 2 * FG:]
    zsrc = jnp.zeros((ZB, FG), jnp.float32)
    for r in range(NMP):
        parts = _sc_segsum(hw0, hw1, hw2, send2, recv2, zsrc)
        args = (h, parts, parts, wua, wub0, wub1, wub2, r2(b_upd1), W_upd2,
                r2(b_upd2), W_msg, r2(b_msg))
        if r < NMP - 1:
            h, hw0, hw1, hw2 = _upd_hw(*args)
        else:
            (h,) = _upd_last(*args)

    logits = _dec_call(h, W_dec1, r2(b_dec1), W_dec2, r2(b_dec2),
                       W_head1, r2(b_head1), W_head2, r2(b_head2))
    return logits[:N].reshape(N, 1, NBERN)


# speedup vs baseline: 8.0917x; 1.2753x over previous
import functools

import jax
import jax.numpy as jnp
from jax import lax
from jax.experimental import pallas as pl
from jax.experimental.pallas import tpu as pltpu
from jax.experimental.pallas import tpu_sc as plsc

N = 100000
E = 1600000
H = 48
EMB = 32
TMAX = 100
NMP = 8
NRAND = 5
NBERN = 2

N2 = 102400
BN = 1600
GRID = N2 // BN

FG = 16
NG = 3
NC = 2
NS = 16
SUB = 125
NSUB = 5
EROWS = E // SUB
ROWS_PT = EROWS // (NC * NS)
NCH = ROWS_PT // NSUB
ACC_PT = N2 // NS
ZB = 1280


def _sc_segsum_body(hw0, hw1, hw2, send2, recv2, zsrc, out,
                    idx_s, idx_r, rows, acc, sem_i, sem_g, sem_s):
    c = lax.axis_index("c")
    s = lax.axis_index("s")
    wid = c * NS + s
    row_base = wid * ROWS_PT

    def _start_idx(pq, r0):
        pltpu.make_async_copy(send2.at[pl.ds(r0, NSUB)], idx_s.at[pq], sem_i).start()
        pltpu.make_async_copy(recv2.at[pl.ds(r0, NSUB)], idx_r.at[pq], sem_i).start()

    def _wait_idx():
        pltpu.make_async_copy(send2.at[pl.ds(row_base, NSUB)], idx_s.at[0], sem_i).wait()
        pltpu.make_async_copy(recv2.at[pl.ds(row_base, NSUB)], idx_r.at[0], sem_i).wait()

    def _start_gathers(hw, pq):
        for j in range(NSUB):
            pltpu.make_async_copy(hw.at[idx_s.at[pq, j]], rows.at[pq, j], sem_g).start()

    def _wait_gathers(hw, pq):
        for j in range(NSUB):
            pltpu.make_async_copy(hw.at[idx_s.at[pq, j]], rows.at[pq, j], sem_g).wait()

    def _start_scatters(pq):
        for j in range(NSUB):
            pltpu.make_async_copy(rows.at[pq, j], acc.at[idx_r.at[pq, j]], sem_s).start(add=True)

    def _wait_scatters(pq):
        for j in range(NSUB):
            pltpu.make_async_copy(rows.at[pq, j], acc.at[idx_r.at[pq, j]], sem_s).wait()

    hws = (hw0, hw1, hw2)
    for g in range(NG):
        hw = hws[g]
        for z in range(ACC_PT // ZB):
            pltpu.sync_copy(zsrc, acc.at[pl.ds(s * ACC_PT + z * ZB, ZB)])
        plsc.subcore_barrier()

        pltpu.sync_copy(send2.at[pl.ds(row_base, NSUB)], idx_s.at[0])
        pltpu.sync_copy(recv2.at[pl.ds(row_base, NSUB)], idx_r.at[0])
        _start_gathers(hw, 0)

        def _chunk(i, _, hw=hw):
            p = lax.rem(i, 2)
            q = 1 - p
            nxt = jnp.minimum(i + 1, NCH - 1)

            @pl.when(i > 0)
            def _():
                _wait_scatters(q)
            _start_idx(q, row_base + nxt * NSUB)
            _wait_gathers(hw, p)
            _start_scatters(p)
            _wait_idx()
            _start_gathers(hw, q)
            return 0
        lax.fori_loop(0, NCH, _chunk, 0)

        _wait_gathers(hw, (NCH - 1) % 2 ^ 1)
        _wait_scatters((NCH - 1) % 2)
        plsc.subcore_barrier()

        pltpu.sync_copy(
            acc.at[pl.ds(s * ACC_PT, ACC_PT)],
            out.at[g, pl.ds(c * N2 + s * ACC_PT, ACC_PT)])
        plsc.subcore_barrier()


_sc_segsum = functools.partial(
    pl.kernel,
    out_type=jax.ShapeDtypeStruct((NG, NC * N2, FG), jnp.float32),
    mesh=plsc.VectorSubcoreMesh(core_axis_name="c", subcore_axis_name="s"),
    compiler_params=pltpu.CompilerParams(use_tc_tiling_on_sc=False),
    scratch_types=[
        pltpu.VMEM((2, NSUB, SUB), jnp.int32),
        pltpu.VMEM((2, NSUB, SUB), jnp.int32),
        pltpu.VMEM((2, NSUB, SUB, FG), jnp.float32),
        pltpu.VMEM_SHARED((N2, FG), jnp.float32),
        pltpu.SemaphoreType.DMA,
        pltpu.SemaphoreType.DMA,
        pltpu.SemaphoreType.DMA,
    ],
)(_sc_segsum_body)


def _full(shape):
    return pl.BlockSpec(shape, lambda i: (0,) * len(shape))


def _rows(bn, width):
    return pl.BlockSpec((bn, width), lambda i: (i, 0))


def _enc_body(xp, temb, rnd, we1, be1, we2, be2, wm, bm,
              h_out, hw0_out, hw1_out, hw2_out):
    x = xp[...]
    feat = jnp.concatenate(
        [(x == 0).astype(jnp.float32), (x == 1).astype(jnp.float32),
         temb[...], rnd[...]], axis=-1)
    u = jnp.dot(feat, we1[...], preferred_element_type=jnp.float32) + be1[...]
    h = jnp.maximum(u, 0.0)
    h = jnp.maximum(jnp.dot(h, we2[...], preferred_element_type=jnp.float32)
                    + be2[...], 0.0)
    hw = jnp.dot(h, wm[...], preferred_element_type=jnp.float32) + bm[...]
    h_out[...] = h
    hw0_out[...] = hw[:, 0:FG]
    hw1_out[...] = hw[:, FG:2 * FG]
    hw2_out[...] = hw[:, 2 * FG:3 * FG]


_enc_call = pl.pallas_call(
    _enc_body,
    grid=(GRID,),
    in_specs=[
        _rows(BN, 1), _rows(BN, EMB), _rows(BN, NRAND),
        _full((2 + EMB + NRAND, H)), _full((1, H)),
        _full((H, H)), _full((1, H)), _full((H, H)), _full((1, H)),
    ],
    out_specs=[_rows(BN, H), _rows(BN, FG), _rows(BN, FG), _rows(BN, FG)],
    out_shape=[
        jax.ShapeDtypeStruct((N2, H), jnp.float32),
        jax.ShapeDtypeStruct((N2, FG), jnp.float32),
        jax.ShapeDtypeStruct((N2, FG), jnp.float32),
        jax.ShapeDtypeStruct((N2, FG), jnp.float32),
    ],
)


def _upd_body(with_hw, h, p0, p1, wu1, bu1, wu2, bu2, wm, bm,
              *outs):
    agg = jnp.concatenate(
        [p0[0, :, :] + p1[0, :, :],
         p0[1, :, :] + p1[1, :, :],
         p0[2, :, :] + p1[2, :, :]], axis=-1)
    hu = jnp.concatenate([h[...], agg], axis=-1)
    u = jnp.dot(hu, wu1[...], preferred_element_type=jnp.float32) + bu1[...]
    u = jnp.maximum(u, 0.0)
    hn = jnp.maximum(jnp.dot(u, wu2[...], preferred_element_type=jnp.float32)
                     + bu2[...], 0.0)
    outs[0][...] = hn
    if with_hw:
        hw = jnp.dot(hn, wm[...], preferred_element_type=jnp.float32) + bm[...]
        outs[1][...] = hw[:, 0:FG]
        outs[2][...] = hw[:, FG:2 * FG]
        outs[3][...] = hw[:, 2 * FG:3 * FG]


def _make_upd(with_hw):
    n_out = 4 if with_hw else 1
    return pl.pallas_call(
        functools.partial(_upd_body, with_hw),
        grid=(GRID,),
        in_specs=[
            _rows(BN, H),
            pl.BlockSpec((NG, BN, FG), lambda i: (0, i, 0)),
            pl.BlockSpec((NG, BN, FG), lambda i: (0, N2 // BN + i, 0)),
            _full((2 * H, H)), _full((1, H)), _full((H, H)), _full((1, H)),
            _full((H, H)), _full((1, H)),
        ],
        out_specs=([_rows(BN, H)] + [_rows(BN, FG)] * 3)[:n_out],
        out_shape=([jax.ShapeDtypeStruct((N2, H), jnp.float32)]
                   + [jax.ShapeDtypeStruct((N2, FG), jnp.float32)] * 3)[:n_out],
    )


_upd_hw = _make_upd(True)
_upd_last = _make_upd(False)


def _dec_body(h, wd1, bd1, wd2, bd2, wh1, bh1, wh2, bh2, out):
    d = jnp.maximum(jnp.dot(h[...], wd1[...], preferred_element_type=jnp.float32)
                    + bd1[...], 0.0)
    d = jnp.dot(d, wd2[...], preferred_element_type=jnp.float32) + bd2[...]
    e = jnp.maximum(jnp.dot(d, wh1[...], preferred_element_type=jnp.float32)
                    + bh1[...], 0.0)
    out[...] = (jnp.dot(e, wh2[...], preferred_element_type=jnp.float32)
                + bh2[...])


_dec_call = pl.pallas_call(
    _dec_body,
    grid=(GRID,),
    in_specs=[
        _rows(BN, H),
        _full((H, H)), _full((1, H)), _full((H, H)), _full((1, H)),
        _full((H, H)), _full((1, H)), _full((H, NBERN)), _full((1, NBERN)),
    ],
    out_specs=_rows(BN, NBERN),
    out_shape=jax.ShapeDtypeStruct((N2, NBERN), jnp.float32),
)


def kernel(x_prev, rand_node_features, t_idx_per_node, edge_index,
           W_enc1, b_enc1, W_enc2, b_enc2, W_msg, b_msg,
           W_upd1, b_upd1, W_upd2, b_upd2, W_dec1, b_dec1, W_dec2, b_dec2,
           W_head1, b_head1, W_head2, b_head2):
    ei = edge_index.astype(jnp.int32)
    send2 = ei[0].reshape(EROWS, SUB)
    recv2 = ei[1].reshape(EROWS, SUB)

    pad = ((0, N2 - N), (0, 0))
    xp = jnp.pad(x_prev, pad)
    rp = jnp.pad(rand_node_features, pad)

    div = jnp.exp(jnp.arange(0, EMB, 2, dtype=jnp.float32)
                  * (-jnp.log(float(TMAX)) / EMB))
    tf = t_idx_per_node.astype(jnp.float32)
    temb = jnp.concatenate([jnp.sin(tf * div), jnp.cos(tf * div)], axis=-1)
    temb = jnp.pad(temb, pad)

    r2 = lambda b: b.reshape(1, -1)
    h, hw0, hw1, hw2 = _enc_call(
        xp, temb, rp,
        W_enc1, r2(b_enc1), W_enc2, r2(b_enc2), W_msg, r2(b_msg))

    zsrc = jnp.zeros((ZB, FG), jnp.float32)
    for r in range(NMP):
        parts = _sc_segsum(hw0, hw1, hw2, send2, recv2, zsrc)
        args = (h, parts, parts, W_upd1, r2(b_upd1), W_upd2,
                r2(b_upd2), W_msg, r2(b_msg))
        if r < NMP - 1:
            h, hw0, hw1, hw2 = _upd_hw(*args)
        else:
            (h,) = _upd_last(*args)

    logits = _dec_call(h, W_dec1, r2(b_dec1), W_dec2, r2(b_dec2),
                       W_head1, r2(b_head1), W_head2, r2(b_head2))
    return logits[:N].reshape(N, 1, NBERN)
